# baseline (speedup 1.0000x reference)
import functools

import jax
import jax.numpy as jnp
from jax import lax
from jax.experimental import pallas as pl
from jax.experimental.pallas import tpu as pltpu
from jax.experimental.pallas import tpu_sc as plsc

_L = 16
_W = 128
_B = 512
_NIDX = _B // _W
_NREG = _B // _L


def _gan_loss_body(vocab, probs_hbm, tgt_hbm, rew_hbm, out_hbm,
                   tgt_vm, rew_vm, col_vm, rows_vm, tot_vm,
                   idx0, idx1, idx2, idx3, sem):
    c = lax.axis_index("c")
    s = lax.axis_index("s")
    idx_refs = (idx0, idx1, idx2, idx3)

    @pl.when(jnp.logical_and(c == 0, s == 0))
    def _work():
        pltpu.sync_copy(tgt_hbm, tgt_vm)
        pltpu.sync_copy(rew_hbm, rew_vm)
        per_idx = _W // _L
        for j in range(_NREG):
            t = tgt_vm[pl.ds(j * _L, _L)]
            row_id = j * _L + lax.iota(jnp.int32, _L)
            flat = row_id * vocab + t
            idx_refs[j // per_idx][pl.ds((j % per_idx) * _L, _L)] = (
                lax.shift_right_logical(flat, 7))
            col_vm[pl.ds(j * _L, _L)] = lax.bitwise_and(flat, _W - 1)
        copies = [
            pltpu.async_copy(probs_hbm.at[idx_refs[k]],
                             rows_vm.at[pl.ds(k * _W, _W)], sem)
            for k in range(_NIDX)
        ]
        for cp in copies:
            cp.wait()
        acc = jnp.zeros((_L,), jnp.float32)
        for j in range(_NREG):
            rid = j * _L + lax.iota(jnp.int32, _L)
            cid = col_vm[pl.ds(j * _L, _L)]
            vals = plsc.load_gather(rows_vm, [rid, cid])
            acc = acc - vals * rew_vm[pl.ds(j * _L, _L)]
        total = jnp.sum(acc)
        tot_vm[...] = lax.broadcast_in_dim(total, (_L,), ())
        pltpu.sync_copy(tot_vm, out_hbm)


@functools.partial(jax.jit, static_argnames=("vocab",))
def _gan_loss(probs2d, targets, rewards, vocab):
    mesh = plsc.VectorSubcoreMesh(core_axis_name="c", subcore_axis_name="s")
    launcher = pl.kernel(
        functools.partial(_gan_loss_body, vocab),
        mesh=mesh,
        out_type=jax.ShapeDtypeStruct((_L,), jnp.float32),
        compiler_params=pltpu.CompilerParams(needs_layout_passes=False),
        scratch_types=[
            pltpu.VMEM((_B,), jnp.int32),
            pltpu.VMEM((_B,), jnp.float32),
            pltpu.VMEM((_B,), jnp.int32),
            pltpu.VMEM((_B, _W), jnp.float32),
            pltpu.VMEM((_L,), jnp.float32),
            pltpu.VMEM((_W,), jnp.int32),
            pltpu.VMEM((_W,), jnp.int32),
            pltpu.VMEM((_W,), jnp.int32),
            pltpu.VMEM((_W,), jnp.int32),
            pltpu.SemaphoreType.DMA,
        ],
    )
    return launcher(probs2d, targets, rewards)


def kernel(probs, targets, rewards):
    vocab = probs.shape[-1]
    probs2d = probs.reshape(-1, _W)
    out = _gan_loss(probs2d, targets, rewards, vocab)
    return out[0]


# speedup vs baseline: 12.8888x; 12.8888x over previous
import functools

import jax
import jax.numpy as jnp
from jax import lax
from jax.experimental import pallas as pl
from jax.experimental.pallas import tpu as pltpu
from jax.experimental.pallas import tpu_sc as plsc

_L = 16
_W = 128
_B = 512
_PER = _B // _L
_NREG = _PER // _L


def _gan_loss_body(probs_hbm, tgt_hbm, rew_hbm, out_hbm, part_hbm,
                   tgt_vm, rew_vm, val_vm, acc_vm, pacc_vm, tot_vm, sem):
    c = lax.axis_index("c")
    s = lax.axis_index("s")

    @pl.when(c == 0)
    def _work():
        base = s * _PER
        pltpu.sync_copy(tgt_hbm.at[pl.ds(base, _PER)], tgt_vm)
        pltpu.sync_copy(rew_hbm.at[pl.ds(base, _PER)], rew_vm)
        tvecs = [tgt_vm[pl.ds(j * _L, _L)] for j in range(_NREG)]
        copies = []
        for k in range(_PER):
            t = tvecs[k // _L][k % _L]
            c0 = pl.multiple_of(
                lax.shift_left(lax.shift_right_logical(t, 7), 7), _W)
            r0 = pl.multiple_of(base + (k & ~7), 8)
            copies.append(pltpu.async_copy(
                probs_hbm.at[pl.ds(r0, 8), pl.ds(c0, _W)],
                val_vm.at[pl.ds(k * 8, 8), :], sem))
        for cp in copies:
            cp.wait()
        acc = jnp.zeros((_L,), jnp.float32)
        sub = lax.bitwise_and(lax.iota(jnp.int32, _L), 7)
        for j in range(_NREG):
            rid = (j * _L + lax.iota(jnp.int32, _L)) * 8 + sub
            cid = lax.bitwise_and(tvecs[j], _W - 1)
            vals = plsc.load_gather(val_vm, [rid, cid])
            acc = acc - vals * rew_vm[pl.ds(j * _L, _L)]
        acc_vm[...] = acc
        pltpu.sync_copy(acc_vm, part_hbm.at[pl.ds(s * _L, _L)])
        plsc.subcore_barrier()

        @pl.when(s == 0)
        def _reduce():
            pltpu.sync_copy(part_hbm, pacc_vm)
            tot = jnp.zeros((_L,), jnp.float32)
            for i in range(_L):
                tot = tot + pacc_vm[pl.ds(i * _L, _L)]
            total = jnp.sum(tot)
            tot_vm[...] = lax.broadcast_in_dim(total, (_L,), ())
            pltpu.sync_copy(tot_vm, out_hbm)


@jax.jit
def _gan_loss(probs2d, targets, rewards):
    mesh = plsc.VectorSubcoreMesh(core_axis_name="c", subcore_axis_name="s")
    launcher = pl.kernel(
        _gan_loss_body,
        mesh=mesh,
        out_type=(jax.ShapeDtypeStruct((_L,), jnp.float32),
                  jax.ShapeDtypeStruct((_L * _L,), jnp.float32)),
        compiler_params=pltpu.CompilerParams(needs_layout_passes=False),
        scratch_types=[
            pltpu.VMEM((_PER,), jnp.int32),
            pltpu.VMEM((_PER,), jnp.float32),
            pltpu.VMEM((_PER * 8, _W), jnp.float32),
            pltpu.VMEM((_L,), jnp.float32),
            pltpu.VMEM((_L * _L,), jnp.float32),
            pltpu.VMEM((_L,), jnp.float32),
            pltpu.SemaphoreType.DMA,
        ],
    )
    out, _parts = launcher(probs2d, targets, rewards)
    return out


def kernel(probs, targets, rewards):
    vocab = probs.shape[-1]
    probs2d = probs.reshape(-1, vocab)
    out = _gan_loss(probs2d, targets, rewards)
    return out[0]


# speedup vs baseline: 13.8657x; 1.0758x over previous
import functools

import jax
import jax.numpy as jnp
from jax import lax
from jax.experimental import pallas as pl
from jax.experimental.pallas import tpu as pltpu
from jax.experimental.pallas import tpu_sc as plsc

_L = 16
_W = 128
_B = 512
_PER = _B // _L
_NREG = _PER // _L


def _gan_loss_body(probs_hbm, tgt_hbm, rew_hbm, out_hbm, part_hbm,
                   tgt_vm, rew_vm, val_vm, acc_vm, pacc_vm, tot_vm, sem):
    c = lax.axis_index("c")
    s = lax.axis_index("s")

    @pl.when(c == 0)
    def _work():
        base = s * _PER
        pltpu.sync_copy(tgt_hbm.at[pl.ds(base, _PER)], tgt_vm)
        pltpu.sync_copy(rew_hbm.at[pl.ds(base, _PER)], rew_vm)
        tvecs = [tgt_vm[pl.ds(j * _L, _L)] for j in range(_NREG)]
        copies = []
        for k in range(_PER):
            t = tvecs[k // _L][k % _L]
            c0 = pl.multiple_of(
                lax.shift_left(lax.shift_right_logical(t, 7), 7), _W)
            r0 = pl.multiple_of(base + (k & ~7), 8)
            copies.append(pltpu.async_copy(
                probs_hbm.at[pl.ds(r0, 8), pl.ds(c0, _W)],
                val_vm.at[pl.ds(k * 8, 8), :], sem))
        for cp in copies:
            cp.wait()
        acc = jnp.zeros((_L,), jnp.float32)
        sub = lax.bitwise_and(lax.iota(jnp.int32, _L), 7)
        for j in range(_NREG):
            rid = (j * _L + lax.iota(jnp.int32, _L)) * 8 + sub
            cid = lax.bitwise_and(tvecs[j], _W - 1)
            vals = plsc.load_gather(val_vm, [rid, cid])
            acc = acc - vals * rew_vm[pl.ds(j * _L, _L)]
        acc_vm[...] = acc
        pltpu.sync_copy(acc_vm, part_hbm.at[pl.ds(s * _L, _L)])
        plsc.subcore_barrier()

        @pl.when(s == 0)
        def _reduce():
            pltpu.sync_copy(part_hbm, pacc_vm)
            tot = jnp.zeros((_L,), jnp.float32)
            for i in range(_L):
                tot = tot + pacc_vm[pl.ds(i * _L, _L)]
            total = jnp.sum(tot)
            tot_vm[...] = lax.broadcast_in_dim(total, (_L,), ())
            pltpu.sync_copy(tot_vm, out_hbm)


@jax.jit
def _gan_loss(probs2d, targets, rewards):
    mesh = plsc.VectorSubcoreMesh(core_axis_name="c", subcore_axis_name="s",
                                  num_cores=1)
    launcher = pl.kernel(
        _gan_loss_body,
        mesh=mesh,
        out_type=(jax.ShapeDtypeStruct((_L,), jnp.float32),
                  jax.ShapeDtypeStruct((_L * _L,), jnp.float32)),
        compiler_params=pltpu.CompilerParams(needs_layout_passes=False),
        scratch_types=[
            pltpu.VMEM((_PER,), jnp.int32),
            pltpu.VMEM((_PER,), jnp.float32),
            pltpu.VMEM((_PER * 8, _W), jnp.float32),
            pltpu.VMEM((_L,), jnp.float32),
            pltpu.VMEM((_L * _L,), jnp.float32),
            pltpu.VMEM((_L,), jnp.float32),
            pltpu.SemaphoreType.DMA,
        ],
    )
    out, _parts = launcher(probs2d, targets, rewards)
    return out


def kernel(probs, targets, rewards):
    vocab = probs.shape[-1]
    probs2d = probs.reshape(-1, vocab)
    out = _gan_loss(probs2d, targets, rewards)
    return out[0]


# speedup vs baseline: 14.1525x; 1.0207x over previous
import jax
import jax.numpy as jnp
from jax import lax
from jax.experimental import pallas as pl
from jax.experimental.pallas import tpu as pltpu
from jax.experimental.pallas import tpu_sc as plsc

_L = 16
_W = 128
_B = 512
_PER = _B // _L
_NREG = _PER // _L
_SOFF = _L


def _gan_loss_body(probs_hbm, tgt_hbm, rew_hbm, out_hbm,
                   tgt_vm, rew_vm, val_vm, acc_vm, pacc_vm, tot_vm,
                   shared, sem):
    s = lax.axis_index("s")
    base = s * _PER
    pltpu.sync_copy(tgt_hbm.at[pl.ds(base, _PER)], tgt_vm)
    pltpu.sync_copy(rew_hbm.at[pl.ds(base, _PER)], rew_vm)
    tvecs = [tgt_vm[pl.ds(j * _L, _L)] for j in range(_NREG)]
    copies = []
    for k in range(_PER):
        t = tvecs[k // _L][k % _L]
        c0 = pl.multiple_of(
            lax.shift_left(lax.shift_right_logical(t, 7), 7), _W)
        r0 = pl.multiple_of(base + (k & ~7), 8)
        copies.append(pltpu.async_copy(
            probs_hbm.at[pl.ds(r0, 8), pl.ds(c0, _W)],
            val_vm.at[pl.ds(k * 8, 8), :], sem))
    for cp in copies:
        cp.wait()
    acc = jnp.zeros((_L,), jnp.float32)
    sub = lax.bitwise_and(lax.iota(jnp.int32, _L), 7)
    for j in range(_NREG):
        rid = (j * _L + lax.iota(jnp.int32, _L)) * 8 + sub
        cid = lax.bitwise_and(tvecs[j], _W - 1)
        vals = plsc.load_gather(val_vm, [rid, cid])
        acc = acc - vals * rew_vm[pl.ds(j * _L, _L)]
    acc_vm[...] = acc
    pltpu.sync_copy(acc_vm, shared.at[_SOFF + s])
    plsc.subcore_barrier()

    @pl.when(s == 0)
    def _reduce():
        pltpu.sync_copy(shared.at[pl.ds(_SOFF, _L)], pacc_vm)
        tot = jnp.zeros((_L,), jnp.float32)
        for i in range(_L):
            tot = tot + pacc_vm[i]
        total = jnp.sum(tot)
        tot_vm[...] = lax.broadcast_in_dim(total, (_L,), ())
        pltpu.sync_copy(tot_vm, out_hbm)


@jax.jit
def _gan_loss(probs2d, targets, rewards):
    mesh = plsc.VectorSubcoreMesh(core_axis_name="c", subcore_axis_name="s",
                                  num_cores=1)
    launcher = pl.kernel(
        _gan_loss_body,
        mesh=mesh,
        out_type=jax.ShapeDtypeStruct((_L,), jnp.float32),
        compiler_params=pltpu.CompilerParams(needs_layout_passes=False),
        scratch_types=[
            pltpu.VMEM((_PER,), jnp.int32),
            pltpu.VMEM((_PER,), jnp.float32),
            pltpu.VMEM((_PER * 8, _W), jnp.float32),
            pltpu.VMEM((_L,), jnp.float32),
            pltpu.VMEM((_L, _L), jnp.float32),
            pltpu.VMEM((_L,), jnp.float32),
            pltpu.VMEM_SHARED((_SOFF + _L, _L), jnp.float32),
            pltpu.SemaphoreType.DMA,
        ],
    )
    return launcher(probs2d, targets, rewards)


def kernel(probs, targets, rewards):
    vocab = probs.shape[-1]
    probs2d = probs.reshape(-1, vocab)
    out = _gan_loss(probs2d, targets, rewards)
    return out[0]


# speedup vs baseline: 14.3624x; 1.0148x over previous
import jax
import jax.numpy as jnp
from jax import lax
from jax.experimental import pallas as pl
from jax.experimental.pallas import tpu as pltpu
from jax.experimental.pallas import tpu_sc as plsc

_L = 16
_W = 128
_B = 512
_PER = _B // _L
_NREG = _PER // _L
_SOFF = _L


def _gan_loss_body(probs_hbm, tgt_hbm, rew_hbm, out_hbm,
                   tgt_vm, rew_vm, val_vm, acc_vm, pacc_vm, tot_vm,
                   shared, sem):
    s = lax.axis_index("s")
    base = s * _PER
    cp_t = pltpu.async_copy(tgt_hbm.at[pl.ds(base, _PER)], tgt_vm, sem)
    cp_r = pltpu.async_copy(rew_hbm.at[pl.ds(base, _PER)], rew_vm, sem)
    cp_t.wait()
    tvecs = [tgt_vm[pl.ds(j * _L, _L)] for j in range(_NREG)]
    copies = []
    for k in range(_PER):
        t = tvecs[k // _L][k % _L]
        c0 = pl.multiple_of(
            lax.shift_left(lax.shift_right_logical(t, 7), 7), _W)
        r0 = pl.multiple_of(base + (k & ~7), 8)
        copies.append(pltpu.async_copy(
            probs_hbm.at[pl.ds(r0, 8), pl.ds(c0, _W)],
            val_vm.at[pl.ds(k * 8, 8), :], sem))
    cp_r.wait()
    for cp in copies:
        cp.wait()
    acc = jnp.zeros((_L,), jnp.float32)
    sub = lax.bitwise_and(lax.iota(jnp.int32, _L), 7)
    for j in range(_NREG):
        rid = (j * _L + lax.iota(jnp.int32, _L)) * 8 + sub
        cid = lax.bitwise_and(tvecs[j], _W - 1)
        vals = plsc.load_gather(val_vm, [rid, cid])
        acc = acc - vals * rew_vm[pl.ds(j * _L, _L)]
    acc_vm[...] = acc
    pltpu.sync_copy(acc_vm, shared.at[_SOFF + s])
    plsc.subcore_barrier()

    @pl.when(s == 0)
    def _reduce():
        pltpu.sync_copy(shared.at[pl.ds(_SOFF, _L)], pacc_vm)
        tot = jnp.zeros((_L,), jnp.float32)
        for i in range(_L):
            tot = tot + pacc_vm[i]
        total = jnp.sum(tot)
        tot_vm[...] = lax.broadcast_in_dim(total, (_L,), ())
        pltpu.sync_copy(tot_vm, out_hbm)


@jax.jit
def _gan_loss(probs2d, targets, rewards):
    mesh = plsc.VectorSubcoreMesh(core_axis_name="c", subcore_axis_name="s",
                                  num_cores=1)
    launcher = pl.kernel(
        _gan_loss_body,
        mesh=mesh,
        out_type=jax.ShapeDtypeStruct((_L,), jnp.float32),
        compiler_params=pltpu.CompilerParams(needs_layout_passes=False),
        scratch_types=[
            pltpu.VMEM((_PER,), jnp.int32),
            pltpu.VMEM((_PER,), jnp.float32),
            pltpu.VMEM((_PER * 8, _W), jnp.float32),
            pltpu.VMEM((_L,), jnp.float32),
            pltpu.VMEM((_L, _L), jnp.float32),
            pltpu.VMEM((_L,), jnp.float32),
            pltpu.VMEM_SHARED((_SOFF + _L, _L), jnp.float32),
            pltpu.SemaphoreType.DMA,
        ],
    )
    return launcher(probs2d, targets, rewards)


def kernel(probs, targets, rewards):
    vocab = probs.shape[-1]
    probs2d = probs.reshape(-1, vocab)
    out = _gan_loss(probs2d, targets, rewards)
    return out[0]


# speedup vs baseline: 14.3834x; 1.0015x over previous
import jax
import jax.numpy as jnp
from jax import lax
from jax.experimental import pallas as pl
from jax.experimental.pallas import tpu as pltpu
from jax.experimental.pallas import tpu_sc as plsc

_L = 16
_W = 128
_B = 512
_PER = _B // _L
_NREG = _PER // _L
_SOFF = _L


def _gan_loss_body(probs_hbm, tgt_hbm, rew_hbm, out_hbm,
                   tgt_vm, rew_vm, val_vm, acc_vm, pacc_vm, tot_vm,
                   shared, sem):
    s = lax.axis_index("s")
    base = s * _PER
    cp_t = pltpu.async_copy(tgt_hbm.at[pl.ds(base, _PER)], tgt_vm, sem)
    cp_r = pltpu.async_copy(rew_hbm.at[pl.ds(base, _PER)], rew_vm, sem)
    cp_t.wait()
    tvecs = [tgt_vm[pl.ds(j * _L, _L)] for j in range(_NREG)]
    for j in range(_NREG):
        tv = tvecs[j]

        @pl.loop(0, _L)
        def _fire(l, tv=tv, j=j):
            lane = lax.broadcast_in_dim(l, (_L,), ())
            t = tv[lane][0]
            k = j * _L + l
            c0 = pl.multiple_of(
                lax.shift_left(lax.shift_right_logical(t, 7), 7), _W)
            r0 = pl.multiple_of(base + lax.bitwise_and(k, -8), 8)
            pltpu.async_copy(
                probs_hbm.at[pl.ds(r0, 8), pl.ds(c0, _W)],
                val_vm.at[pl.ds(k * 8, 8), :], sem)

    cp_r.wait()

    @pl.loop(0, _PER)
    def _drain(l):
        pltpu.make_async_copy(
            probs_hbm.at[pl.ds(0, 8), pl.ds(0, _W)],
            val_vm.at[pl.ds(0, 8), :], sem).wait()
    acc = jnp.zeros((_L,), jnp.float32)
    sub = lax.bitwise_and(lax.iota(jnp.int32, _L), 7)
    for j in range(_NREG):
        rid = (j * _L + lax.iota(jnp.int32, _L)) * 8 + sub
        cid = lax.bitwise_and(tvecs[j], _W - 1)
        vals = plsc.load_gather(val_vm, [rid, cid])
        acc = acc - vals * rew_vm[pl.ds(j * _L, _L)]
    acc_vm[...] = acc
    pltpu.sync_copy(acc_vm, shared.at[_SOFF + s])
    plsc.subcore_barrier()

    @pl.when(s == 0)
    def _reduce():
        pltpu.sync_copy(shared.at[pl.ds(_SOFF, _L)], pacc_vm)
        tot = jnp.zeros((_L,), jnp.float32)
        for i in range(_L):
            tot = tot + pacc_vm[i]
        total = jnp.sum(tot)
        tot_vm[...] = lax.broadcast_in_dim(total, (_L,), ())
        pltpu.sync_copy(tot_vm, out_hbm)


@jax.jit
def _gan_loss(probs2d, targets, rewards):
    mesh = plsc.VectorSubcoreMesh(core_axis_name="c", subcore_axis_name="s",
                                  num_cores=1)
    launcher = pl.kernel(
        _gan_loss_body,
        mesh=mesh,
        out_type=jax.ShapeDtypeStruct((_L,), jnp.float32),
        compiler_params=pltpu.CompilerParams(needs_layout_passes=False),
        scratch_types=[
            pltpu.VMEM((_PER,), jnp.int32),
            pltpu.VMEM((_PER,), jnp.float32),
            pltpu.VMEM((_PER * 8, _W), jnp.float32),
            pltpu.VMEM((_L,), jnp.float32),
            pltpu.VMEM((_L, _L), jnp.float32),
            pltpu.VMEM((_L,), jnp.float32),
            pltpu.VMEM_SHARED((_SOFF + _L, _L), jnp.float32),
            pltpu.SemaphoreType.DMA,
        ],
    )
    return launcher(probs2d, targets, rewards)


def kernel(probs, targets, rewards):
    vocab = probs.shape[-1]
    probs2d = probs.reshape(-1, vocab)
    out = _gan_loss(probs2d, targets, rewards)
    return out[0]
